# baseline (speedup 1.0000x reference)
import functools

import jax
import jax.numpy as jnp
from jax import lax
from jax.experimental import pallas as pl

DIM = 2048
N_EXPERTS = 64
N_GROUPS = 8
GROUP_SIZE = N_EXPERTS // N_GROUPS
TOPK_GROUPS = 4
TOP_K = 8
ROUTE_SCALE = 2.5

NEG_INF = float("-inf")


def _router_block(x_ref, w_ref, out_w_ref, out_i_ref):
    m = x_ref.shape[0]
    logits = lax.dot_general(
        w_ref[...], x_ref[...],
        (((1,), (1,)), ((), ())),
        preferred_element_type=jnp.float32,
    )
    scores = jax.nn.sigmoid(logits)

    gmax = jnp.concatenate(
        [jnp.max(scores[g * GROUP_SIZE:(g + 1) * GROUP_SIZE, :], axis=0,
                 keepdims=True)
         for g in range(N_GROUPS)], axis=0)

    gidx = lax.broadcasted_iota(jnp.int32, (N_GROUPS, m), 0).astype(
        jnp.float32)
    rank = jnp.zeros((N_GROUPS, m), dtype=jnp.float32)
    for gp in range(N_GROUPS):
        v = gmax[gp:gp + 1, :]
        beats = (v > gmax) | ((v == gmax) & (gp < gidx))
        rank = rank + jnp.where(beats, 1.0, 0.0)

    penalty = jnp.where(rank < TOPK_GROUPS, 0.0, NEG_INF)
    penalty_lane = jnp.concatenate(
        [jnp.broadcast_to(penalty[g:g + 1, :], (GROUP_SIZE, m))
         for g in range(N_GROUPS)], axis=0)
    masked = scores + penalty_lane

    lane = lax.broadcasted_iota(jnp.int32, (N_EXPERTS, m), 0).astype(
        jnp.float32)
    cur = masked
    vals, idxs = [], []
    for _ in range(TOP_K):
        mx = jnp.max(cur, axis=0, keepdims=True)
        sel = jnp.min(jnp.where(cur == mx, lane, float(N_EXPERTS)),
                      axis=0, keepdims=True)
        vals.append(mx)
        idxs.append(sel)
        cur = jnp.where(lane == sel, NEG_INF, cur)
    vals = jnp.concatenate(vals, axis=0)
    idxs = jnp.concatenate(idxs, axis=0)

    wsum = jnp.sum(vals, axis=0, keepdims=True)
    out_w_ref[...] = (vals / wsum * ROUTE_SCALE).T
    out_i_ref[...] = idxs.astype(jnp.int32).T


@functools.partial(jax.jit, static_argnames=("block_m", "interpret"))
def _run(x, weight, block_m=2048, interpret=False):
    n = x.shape[0]
    grid = (n // block_m,)
    out_w_t, out_i_t = pl.pallas_call(
        _router_block,
        grid=grid,
        in_specs=[
            pl.BlockSpec((block_m, DIM), lambda i: (i, 0)),
            pl.BlockSpec((N_EXPERTS, DIM), lambda i: (0, 0)),
        ],
        out_specs=[
            pl.BlockSpec((block_m, TOP_K), lambda i: (i, 0)),
            pl.BlockSpec((block_m, TOP_K), lambda i: (i, 0)),
        ],
        out_shape=[
            jax.ShapeDtypeStruct((n, TOP_K), jnp.float32),
            jax.ShapeDtypeStruct((n, TOP_K), jnp.int32),
        ],
        interpret=interpret,
    )(x, weight)
    return out_w_t, out_i_t


def kernel(x, weight):
    return tuple(_run(x, weight))


# speedup vs baseline: 1.3486x; 1.3486x over previous
import functools

import jax
import jax.numpy as jnp
from jax import lax
from jax.experimental import pallas as pl

DIM = 2048
N_EXPERTS = 64
N_GROUPS = 8
GROUP_SIZE = N_EXPERTS // N_GROUPS
TOPK_GROUPS = 4
TOP_K = 8
ROUTE_SCALE = 2.5

NEG_INF = float("-inf")


def _router_block(x_ref, w_ref, out_w_ref, out_i_ref):
    m = x_ref.shape[0]
    logits = lax.dot_general(
        w_ref[...], x_ref[...],
        (((1,), (1,)), ((), ())),
        preferred_element_type=jnp.float32,
    )
    scores = jax.nn.sigmoid(logits)

    gmax = jnp.concatenate(
        [jnp.max(scores[g * GROUP_SIZE:(g + 1) * GROUP_SIZE, :], axis=0,
                 keepdims=True)
         for g in range(N_GROUPS)], axis=0)

    gidx = lax.broadcasted_iota(jnp.int32, (N_GROUPS, m), 0).astype(
        jnp.float32)
    rank = jnp.zeros((N_GROUPS, m), dtype=jnp.float32)
    for gp in range(N_GROUPS):
        v = gmax[gp:gp + 1, :]
        beats = (v > gmax) | ((v == gmax) & (gp < gidx))
        rank = rank + jnp.where(beats, 1.0, 0.0)

    penalty = jnp.where(rank < TOPK_GROUPS, 0.0, NEG_INF)
    penalty_lane = jnp.concatenate(
        [jnp.broadcast_to(penalty[g:g + 1, :], (GROUP_SIZE, m))
         for g in range(N_GROUPS)], axis=0)
    masked = scores + penalty_lane

    lane = lax.broadcasted_iota(jnp.int32, (N_EXPERTS, m), 0).astype(
        jnp.float32)
    cur = masked
    vals, idxs = [], []
    for _ in range(TOP_K):
        mx = jnp.max(cur, axis=0, keepdims=True)
        sel = jnp.min(jnp.where(cur == mx, lane, float(N_EXPERTS)),
                      axis=0, keepdims=True)
        vals.append(mx)
        idxs.append(sel)
        cur = jnp.where(lane == sel, NEG_INF, cur)
    vals = jnp.concatenate(vals, axis=0)
    idxs = jnp.concatenate(idxs, axis=0)

    wsum = jnp.sum(vals, axis=0, keepdims=True)
    out_w_ref[...] = vals / wsum * ROUTE_SCALE
    out_i_ref[...] = idxs.astype(jnp.int32)


@functools.partial(jax.jit, static_argnames=("block_m",))
def _run(x, weight, block_m=2048):
    n = x.shape[0]
    grid = (n // block_m,)
    out_w_t, out_i_t = pl.pallas_call(
        _router_block,
        grid=grid,
        in_specs=[
            pl.BlockSpec((block_m, DIM), lambda i: (i, 0)),
            pl.BlockSpec((N_EXPERTS, DIM), lambda i: (0, 0)),
        ],
        out_specs=[
            pl.BlockSpec((TOP_K, block_m), lambda i: (0, i)),
            pl.BlockSpec((TOP_K, block_m), lambda i: (0, i)),
        ],
        out_shape=[
            jax.ShapeDtypeStruct((TOP_K, n), jnp.float32),
            jax.ShapeDtypeStruct((TOP_K, n), jnp.int32),
        ],
    )(x, weight)
    return out_w_t.T, out_i_t.T


def kernel(x, weight):
    return tuple(_run(x, weight))


# speedup vs baseline: 1.3490x; 1.0003x over previous
import functools

import jax
import jax.numpy as jnp
from jax import lax
from jax.experimental import pallas as pl
from jax.experimental.pallas import tpu as pltpu

DIM = 2048
N_EXPERTS = 64
N_GROUPS = 8
GROUP_SIZE = N_EXPERTS // N_GROUPS
TOPK_GROUPS = 4
TOP_K = 8
ROUTE_SCALE = 2.5

NEG_INF = float("-inf")


def _router_block(x_ref, w_ref, out_w_ref, out_i_ref):
    m = x_ref.shape[0]
    logits = lax.dot_general(
        w_ref[...], x_ref[...],
        (((1,), (1,)), ((), ())),
        preferred_element_type=jnp.float32,
    )
    scores = jax.nn.sigmoid(logits)

    gmax = jnp.concatenate(
        [jnp.max(scores[g * GROUP_SIZE:(g + 1) * GROUP_SIZE, :], axis=0,
                 keepdims=True)
         for g in range(N_GROUPS)], axis=0)

    gidx = lax.broadcasted_iota(jnp.int32, (N_GROUPS, m), 0).astype(
        jnp.float32)
    rank = jnp.zeros((N_GROUPS, m), dtype=jnp.float32)
    for gp in range(N_GROUPS):
        v = gmax[gp:gp + 1, :]
        beats = (v > gmax) | ((v == gmax) & (gp < gidx))
        rank = rank + jnp.where(beats, 1.0, 0.0)

    penalty = jnp.where(rank < TOPK_GROUPS, 0.0, NEG_INF)
    penalty_lane = jnp.concatenate(
        [jnp.broadcast_to(penalty[g:g + 1, :], (GROUP_SIZE, m))
         for g in range(N_GROUPS)], axis=0)
    masked = scores + penalty_lane

    lane = lax.broadcasted_iota(jnp.int32, (N_EXPERTS, m), 0).astype(
        jnp.float32)
    cur = masked
    vals, idxs = [], []
    for _ in range(TOP_K):
        mx = jnp.max(cur, axis=0, keepdims=True)
        sel = jnp.min(jnp.where(cur == mx, lane, float(N_EXPERTS)),
                      axis=0, keepdims=True)
        vals.append(mx)
        idxs.append(sel)
        cur = jnp.where(lane == sel, NEG_INF, cur)
    vals = jnp.concatenate(vals, axis=0)
    idxs = jnp.concatenate(idxs, axis=0)

    wsum = jnp.sum(vals, axis=0, keepdims=True)
    out_w_ref[...] = vals / wsum * ROUTE_SCALE
    out_i_ref[...] = idxs.astype(jnp.int32)


@functools.partial(jax.jit, static_argnames=("block_m",))
def _run(x, weight, block_m=2048):
    n = x.shape[0]
    grid = (n // block_m,)
    out_w_t, out_i_t = pl.pallas_call(
        _router_block,
        grid=grid,
        in_specs=[
            pl.BlockSpec((block_m, DIM), lambda i: (i, 0)),
            pl.BlockSpec((N_EXPERTS, DIM), lambda i: (0, 0)),
        ],
        out_specs=[
            pl.BlockSpec((TOP_K, block_m), lambda i: (0, i)),
            pl.BlockSpec((TOP_K, block_m), lambda i: (0, i)),
        ],
        out_shape=[
            jax.ShapeDtypeStruct((TOP_K, n), jnp.float32),
            jax.ShapeDtypeStruct((TOP_K, n), jnp.int32),
        ],
        compiler_params=pltpu.CompilerParams(
            dimension_semantics=("parallel",)),
    )(x, weight)
    return out_w_t.T, out_i_t.T


def kernel(x, weight):
    return tuple(_run(x, weight))
